# baseline (speedup 1.0000x reference)
import functools

import jax
import jax.numpy as jnp
from jax import lax
from jax.experimental import pallas as pl
from jax.experimental.pallas import tpu as pltpu
from jax.experimental.pallas import tpu_sc as plsc

NODE_SIZE = 100000
PATH_SIZE = 64
EMBED_DIM = 128
NUM_ETYPES = 4
NEI = 16
BATCH = 1024

NC = 2
NS = 16
NW = NC * NS
BPW = BATCH // NW
ROWS = NUM_ETYPES * NEI
KB = 8
NWIN = 4


def _sc_body(nbrT_hbm, sidx_hbm, eidx_hbm, pidx_hbm, estart_hbm, eend_hbm,
             epath_hbm, agg_hbm, eemb_hbm, praw_hbm,
             idx_v, eidx_v, pidx_v, nbr_vs, sel_vs, rows_vs, out_v,
             eemb_v, pemb_v, wsems, sems, sem_m, sem_p):
    wid = lax.axis_index("s") * NC + lax.axis_index("c")
    base = wid * BPW

    pltpu.sync_copy(sidx_hbm.at[pl.ds(base, BPW)], idx_v)
    pltpu.sync_copy(eidx_hbm.at[pl.ds(base, BPW)], eidx_v)
    pltpu.sync_copy(pidx_hbm.at[pl.ds(base, BPW)], pidx_v)
    pltpu.async_copy(eend_hbm.at[eidx_v], eemb_v, sem_m)
    pltpu.async_copy(epath_hbm.at[pidx_v], pemb_v, sem_p)

    def accum(j, g):
        for e in range(NUM_ETYPES):
            for c in range(EMBED_DIM // 16):
                sl = pl.ds(c * 16, 16)
                vals = [rows_vs[g][e * NEI + r, sl] for r in range(NEI)]
                while len(vals) > 1:
                    vals = [vals[i] + vals[i + 1]
                            for i in range(0, len(vals), 2)]
                out_v[e, j, sl] = vals[0] * (1.0 / NEI)

    iota16 = lax.broadcasted_iota(jnp.int32, (16,), 0)

    def fire_win(j, w):
        sjv = plsc.load_gather(idx_v, [jnp.full((16,), j, jnp.int32)])
        t128 = pl.multiple_of(
            lax.shift_left(lax.shift_right_logical(sjv[0], 7), 7), 128)
        pltpu.async_copy(nbrT_hbm.at[:, :, pl.ds(t128, 128)],
                         nbr_vs[w], wsems[w])
        return sjv

    def sel_fire_rows(g, w, sjv):
        pltpu.make_async_copy(
            nbrT_hbm.at[:, :, pl.ds(0, 128)], nbr_vs[w], wsems[w]).wait()
        q = sjv & 127
        for e in range(NUM_ETYPES):
            ids16 = plsc.load_gather(
                nbr_vs[w], [jnp.full((16,), e, jnp.int32), iota16, q])
            sel_vs[g][pl.ds(e * NEI, 16)] = ids16
        pltpu.async_copy(estart_hbm.at[sel_vs[g]], rows_vs[g], sems[g])

    def body(h, carry):
        j0 = h * KB
        sjvs = [None] * KB
        for g in range(NWIN):
            sjvs[g] = fire_win(j0 + g, g)
        for g in range(NWIN):
            sel_fire_rows(g, g, sjvs[g])
            sjvs[NWIN + g] = fire_win(j0 + NWIN + g, g)
        for g in range(NWIN, KB):
            sel_fire_rows(g, g - NWIN, sjvs[g])
        for g in range(KB):
            pltpu.make_async_copy(estart_hbm.at[sel_vs[g]], rows_vs[g],
                                  sems[g]).wait()
            accum(j0 + g, g)
        return carry

    lax.fori_loop(0, BPW // KB, body, 0)

    for e in range(NUM_ETYPES):
        pltpu.sync_copy(out_v.at[e], agg_hbm.at[e, pl.ds(base, BPW)])
    pltpu.make_async_copy(eend_hbm.at[eidx_v], eemb_v, sem_m).wait()
    pltpu.sync_copy(eemb_v, eemb_hbm.at[pl.ds(base, BPW)])
    pltpu.make_async_copy(epath_hbm.at[pidx_v], pemb_v, sem_p).wait()
    pltpu.sync_copy(pemb_v, praw_hbm.at[pl.ds(base, BPW)])


def _sc_entry(nbrT_hbm, sidx_hbm, eidx_hbm, pidx_hbm, estart_hbm, eend_hbm,
              epath_hbm, agg_hbm, eemb_hbm, praw_hbm,
              idx_v, eidx_v, pidx_v,
              n0, n1, n2, n3,
              c0, c1, c2, c3, c4, c5, c6, c7,
              r0, r1, r2, r3, r4, r5, r6, r7,
              out_v, eemb_v, pemb_v, w0, w1, w2, w3,
              d0, d1, d2, d3, d4, d5, d6, d7, sem_m, sem_p):
    _sc_body(nbrT_hbm, sidx_hbm, eidx_hbm, pidx_hbm, estart_hbm, eend_hbm,
             epath_hbm, agg_hbm, eemb_hbm, praw_hbm,
             idx_v, eidx_v, pidx_v, (n0, n1, n2, n3),
             (c0, c1, c2, c3, c4, c5, c6, c7),
             (r0, r1, r2, r3, r4, r5, r6, r7),
             out_v, eemb_v, pemb_v, (w0, w1, w2, w3),
             (d0, d1, d2, d3, d4, d5, d6, d7), sem_m, sem_p)


_sc_gather = functools.partial(
    pl.kernel,
    out_type=(
        jax.ShapeDtypeStruct((NUM_ETYPES, BATCH, EMBED_DIM), jnp.float32),
        jax.ShapeDtypeStruct((BATCH, EMBED_DIM), jnp.float32),
        jax.ShapeDtypeStruct((BATCH, EMBED_DIM), jnp.float32),
    ),
    mesh=plsc.VectorSubcoreMesh(
        core_axis_name="c", subcore_axis_name="s", num_cores=NC,
        num_subcores=NS),
    compiler_params=pltpu.CompilerParams(needs_layout_passes=False),
    scratch_types=(
        [pltpu.VMEM((BPW,), jnp.int32)] * 3
        + [pltpu.VMEM((NUM_ETYPES, NEI, 128), jnp.int32)] * NWIN
        + [pltpu.VMEM((ROWS,), jnp.int32)] * KB
        + [pltpu.VMEM((ROWS, EMBED_DIM), jnp.float32)] * KB
        + [pltpu.VMEM((NUM_ETYPES, BPW, EMBED_DIM), jnp.float32)]
        + [pltpu.VMEM((BPW, EMBED_DIM), jnp.float32)] * 2
        + [pltpu.SemaphoreType.DMA] * (NWIN + KB + 2)
    ),
)(_sc_entry)


def _tc_body(agg_ref, eemb_ref, praw_ref, W1_ref, b1_ref, W2_ref, b2_ref,
             out_ref):
    f32 = jnp.float32
    hi = lax.Precision.HIGHEST
    W1 = W1_ref[...]
    b1 = b1_ref[...]
    acc = jnp.broadcast_to(b2_ref[...], (BATCH, EMBED_DIM))
    for e in range(NUM_ETYPES):
        h = lax.dot_general(agg_ref[e], W1, (((1,), (0,)), ((), ())),
                            precision=hi, preferred_element_type=f32) + b1
        W2e = W2_ref[pl.ds(e * EMBED_DIM, EMBED_DIM), :]
        acc = acc + lax.dot_general(h, W2e, (((1,), (0,)), ((), ())),
                                    precision=hi, preferred_element_type=f32)
    p = jax.nn.sigmoid(praw_ref[...])
    m = acc * eemb_ref[...] * p
    out_ref[...] = jax.nn.sigmoid(jnp.sum(m, axis=1, keepdims=True))


_tc_dense = pl.pallas_call(
    _tc_body,
    out_shape=jax.ShapeDtypeStruct((BATCH, 1), jnp.float32),
)


def kernel(neighbors, start_node, end_node, path, embeds_start, embeds_end,
           embeds_path, W1, b1, W2, b2):
    nbrT = jnp.transpose(neighbors, (1, 2, 0))
    agg, eemb, praw = _sc_gather(
        nbrT, start_node.astype(jnp.int32), end_node.astype(jnp.int32),
        path.astype(jnp.int32), embeds_start, embeds_end, embeds_path)
    out = _tc_dense(agg, eemb, praw, W1, b1.reshape(1, EMBED_DIM), W2,
                    b2.reshape(1, EMBED_DIM))
    return out.reshape(BATCH)


# speedup vs baseline: 1.0261x; 1.0261x over previous
import functools

import jax
import jax.numpy as jnp
from jax import lax
from jax.experimental import pallas as pl
from jax.experimental.pallas import tpu as pltpu
from jax.experimental.pallas import tpu_sc as plsc

NODE_SIZE = 100000
PATH_SIZE = 64
EMBED_DIM = 128
NUM_ETYPES = 4
NEI = 16
BATCH = 1024

NC = 2
NS = 16
NW = NC * NS
BPW = BATCH // NW
ROWS = NUM_ETYPES * NEI
KB = 4


def _sc_body(nbrT_hbm, sidx_hbm, eidx_hbm, pidx_hbm, estart_hbm, eend_hbm,
             epath_hbm, agg_hbm, eemb_hbm, praw_hbm,
             idx_v, eidx_v, pidx_v, nbr_vs, sel_vs, rows_vs, out_v,
             eemb_v, pemb_v, wsems, sems, sem_m, sem_p):
    wid = lax.axis_index("s") * NC + lax.axis_index("c")
    base = wid * BPW

    pltpu.sync_copy(sidx_hbm.at[pl.ds(base, BPW)], idx_v)
    pltpu.sync_copy(eidx_hbm.at[pl.ds(base, BPW)], eidx_v)
    pltpu.sync_copy(pidx_hbm.at[pl.ds(base, BPW)], pidx_v)
    pltpu.async_copy(eend_hbm.at[eidx_v], eemb_v, sem_m)
    pltpu.async_copy(epath_hbm.at[pidx_v], pemb_v, sem_p)

    def accum(j, g):
        for e in range(NUM_ETYPES):
            for c in range(EMBED_DIM // 16):
                sl = pl.ds(c * 16, 16)
                vals = [rows_vs[g][e * NEI + r, sl] for r in range(NEI)]
                while len(vals) > 1:
                    vals = [vals[i] + vals[i + 1]
                            for i in range(0, len(vals), 2)]
                out_v[e, j, sl] = vals[0] * (1.0 / NEI)

    iota16 = lax.broadcasted_iota(jnp.int32, (16,), 0)

    def body(h, carry):
        j0 = h * KB
        sjvs = []
        for g in range(KB):
            sjv = plsc.load_gather(
                idx_v, [jnp.full((16,), j0 + g, jnp.int32)])
            sjvs.append(sjv)
            t128 = pl.multiple_of(
                lax.shift_left(lax.shift_right_logical(sjv[0], 7), 7), 128)
            pltpu.async_copy(nbrT_hbm.at[:, :, pl.ds(t128, 128)],
                             nbr_vs[g], wsems[g])
        for g in range(KB):
            pltpu.make_async_copy(
                nbrT_hbm.at[:, :, pl.ds(0, 128)], nbr_vs[g], wsems[g]).wait()
            q = sjvs[g] & 127
            for e in range(NUM_ETYPES):
                ids16 = plsc.load_gather(
                    nbr_vs[g], [jnp.full((16,), e, jnp.int32), iota16, q])
                sel_vs[g][pl.ds(e * NEI, 16)] = ids16
            pltpu.async_copy(estart_hbm.at[sel_vs[g]], rows_vs[g], sems[g])
        for g in range(KB):
            pltpu.make_async_copy(estart_hbm.at[sel_vs[g]], rows_vs[g],
                                  sems[g]).wait()
            accum(j0 + g, g)
        return carry

    lax.fori_loop(0, BPW // KB, body, 0)

    for e in range(NUM_ETYPES):
        pltpu.sync_copy(out_v.at[e], agg_hbm.at[e, pl.ds(base, BPW)])
    pltpu.make_async_copy(eend_hbm.at[eidx_v], eemb_v, sem_m).wait()
    pltpu.sync_copy(eemb_v, eemb_hbm.at[pl.ds(base, BPW)])
    pltpu.make_async_copy(epath_hbm.at[pidx_v], pemb_v, sem_p).wait()
    pltpu.sync_copy(pemb_v, praw_hbm.at[pl.ds(base, BPW)])


def _sc_entry(nbrT_hbm, sidx_hbm, eidx_hbm, pidx_hbm, estart_hbm, eend_hbm,
              epath_hbm, agg_hbm, eemb_hbm, praw_hbm,
              idx_v, eidx_v, pidx_v,
              n0, n1, n2, n3,
              c0, c1, c2, c3,
              r0, r1, r2, r3,
              out_v, eemb_v, pemb_v,
              w0, w1, w2, w3,
              d0, d1, d2, d3, sem_m, sem_p):
    _sc_body(nbrT_hbm, sidx_hbm, eidx_hbm, pidx_hbm, estart_hbm, eend_hbm,
             epath_hbm, agg_hbm, eemb_hbm, praw_hbm,
             idx_v, eidx_v, pidx_v, (n0, n1, n2, n3), (c0, c1, c2, c3),
             (r0, r1, r2, r3),
             out_v, eemb_v, pemb_v, (w0, w1, w2, w3), (d0, d1, d2, d3),
             sem_m, sem_p)


_sc_gather = functools.partial(
    pl.kernel,
    out_type=(
        jax.ShapeDtypeStruct((NUM_ETYPES, BATCH, EMBED_DIM), jnp.float32),
        jax.ShapeDtypeStruct((BATCH, EMBED_DIM), jnp.float32),
        jax.ShapeDtypeStruct((BATCH, EMBED_DIM), jnp.float32),
    ),
    mesh=plsc.VectorSubcoreMesh(
        core_axis_name="c", subcore_axis_name="s", num_cores=NC,
        num_subcores=NS),
    compiler_params=pltpu.CompilerParams(needs_layout_passes=False),
    scratch_types=(
        [pltpu.VMEM((BPW,), jnp.int32)] * 3
        + [pltpu.VMEM((NUM_ETYPES, NEI, 128), jnp.int32)] * KB
        + [pltpu.VMEM((ROWS,), jnp.int32)] * KB
        + [pltpu.VMEM((ROWS, EMBED_DIM), jnp.float32)] * KB
        + [pltpu.VMEM((NUM_ETYPES, BPW, EMBED_DIM), jnp.float32)]
        + [pltpu.VMEM((BPW, EMBED_DIM), jnp.float32)] * 2
        + [pltpu.SemaphoreType.DMA] * (2 * KB + 2)
    ),
)(_sc_entry)


def _tc_body(agg_ref, eemb_ref, praw_ref, W1_ref, b1_ref, W2_ref, b2_ref,
             out_ref):
    f32 = jnp.float32
    hi = lax.Precision.HIGHEST
    W1 = W1_ref[...]
    b1 = b1_ref[...]
    acc = jnp.broadcast_to(b2_ref[...], (BATCH, EMBED_DIM))
    for e in range(NUM_ETYPES):
        h = lax.dot_general(agg_ref[e], W1, (((1,), (0,)), ((), ())),
                            precision=hi, preferred_element_type=f32) + b1
        W2e = W2_ref[pl.ds(e * EMBED_DIM, EMBED_DIM), :]
        acc = acc + lax.dot_general(h, W2e, (((1,), (0,)), ((), ())),
                                    precision=hi, preferred_element_type=f32)
    p = jax.nn.sigmoid(praw_ref[...])
    m = acc * eemb_ref[...] * p
    out_ref[...] = jax.nn.sigmoid(jnp.sum(m, axis=1, keepdims=True))


_tc_dense = pl.pallas_call(
    _tc_body,
    out_shape=jax.ShapeDtypeStruct((BATCH, 1), jnp.float32),
)


def kernel(neighbors, start_node, end_node, path, embeds_start, embeds_end,
           embeds_path, W1, b1, W2, b2):
    nbrT = jnp.transpose(neighbors, (1, 2, 0))
    agg, eemb, praw = _sc_gather(
        nbrT, start_node.astype(jnp.int32), end_node.astype(jnp.int32),
        path.astype(jnp.int32), embeds_start, embeds_end, embeds_path)
    out = _tc_dense(agg, eemb, praw, W1, b1.reshape(1, EMBED_DIM), W2,
                    b2.reshape(1, EMBED_DIM))
    return out.reshape(BATCH)


# speedup vs baseline: 1.0348x; 1.0085x over previous
import functools

import jax
import jax.numpy as jnp
from jax import lax
from jax.experimental import pallas as pl
from jax.experimental.pallas import tpu as pltpu
from jax.experimental.pallas import tpu_sc as plsc

NODE_SIZE = 100000
PATH_SIZE = 64
EMBED_DIM = 128
NUM_ETYPES = 4
NEI = 16
BATCH = 1024

NC = 2
NS = 16
NW = NC * NS
BPW = BATCH // NW
ROWS = NUM_ETYPES * NEI
KB = 4


def _sc_body(nbrT_hbm, sidx_hbm, eidx_hbm, pidx_hbm, estart_hbm, eend_hbm,
             epath_hbm, agg_hbm, eemb_hbm, praw_hbm,
             idx_v, eidx_v, pidx_v, nbr_vs, sel_vs, rows_vs, out_v,
             eemb_v, pemb_v, wsems, sems, sem_m, sem_p):
    wid = lax.axis_index("s") * NC + lax.axis_index("c")
    base = wid * BPW

    pltpu.async_copy(sidx_hbm.at[pl.ds(base, BPW)], idx_v, wsems[0])
    pltpu.async_copy(eidx_hbm.at[pl.ds(base, BPW)], eidx_v, wsems[1])
    pltpu.async_copy(pidx_hbm.at[pl.ds(base, BPW)], pidx_v, wsems[2])
    pltpu.make_async_copy(eidx_hbm.at[pl.ds(base, BPW)], eidx_v,
                          wsems[1]).wait()
    pltpu.async_copy(eend_hbm.at[eidx_v], eemb_v, sem_m)
    pltpu.make_async_copy(pidx_hbm.at[pl.ds(base, BPW)], pidx_v,
                          wsems[2]).wait()
    pltpu.async_copy(epath_hbm.at[pidx_v], pemb_v, sem_p)
    pltpu.make_async_copy(sidx_hbm.at[pl.ds(base, BPW)], idx_v,
                          wsems[0]).wait()

    def accum(j, g):
        for e in range(NUM_ETYPES):
            for c in range(EMBED_DIM // 16):
                sl = pl.ds(c * 16, 16)
                vals = [rows_vs[g][e * NEI + r, sl] for r in range(NEI)]
                while len(vals) > 1:
                    vals = [vals[i] + vals[i + 1]
                            for i in range(0, len(vals), 2)]
                out_v[e, j, sl] = vals[0] * (1.0 / NEI)

    iota16 = lax.broadcasted_iota(jnp.int32, (16,), 0)

    def body(h, carry):
        j0 = h * KB
        sjvs = []
        for g in range(KB):
            sjv = plsc.load_gather(
                idx_v, [jnp.full((16,), j0 + g, jnp.int32)])
            sjvs.append(sjv)
            t128 = pl.multiple_of(
                lax.shift_left(lax.shift_right_logical(sjv[0], 7), 7), 128)
            pltpu.async_copy(nbrT_hbm.at[:, :, pl.ds(t128, 128)],
                             nbr_vs[g], wsems[g])
        for g in range(KB):
            pltpu.make_async_copy(
                nbrT_hbm.at[:, :, pl.ds(0, 128)], nbr_vs[g], wsems[g]).wait()
            q = sjvs[g] & 127
            for e in range(NUM_ETYPES):
                ids16 = plsc.load_gather(
                    nbr_vs[g], [jnp.full((16,), e, jnp.int32), iota16, q])
                sel_vs[g][pl.ds(e * NEI, 16)] = ids16
            pltpu.async_copy(estart_hbm.at[sel_vs[g]], rows_vs[g], sems[g])
        for g in range(KB):
            pltpu.make_async_copy(estart_hbm.at[sel_vs[g]], rows_vs[g],
                                  sems[g]).wait()
            accum(j0 + g, g)
        return carry

    lax.fori_loop(0, BPW // KB, body, 0)

    for e in range(NUM_ETYPES):
        pltpu.async_copy(out_v.at[e], agg_hbm.at[e, pl.ds(base, BPW)],
                         wsems[e])
    pltpu.make_async_copy(eend_hbm.at[eidx_v], eemb_v, sem_m).wait()
    pltpu.async_copy(eemb_v, eemb_hbm.at[pl.ds(base, BPW)], sems[0])
    pltpu.make_async_copy(epath_hbm.at[pidx_v], pemb_v, sem_p).wait()
    pltpu.async_copy(pemb_v, praw_hbm.at[pl.ds(base, BPW)], sems[1])
    for e in range(NUM_ETYPES):
        pltpu.make_async_copy(out_v.at[e], agg_hbm.at[e, pl.ds(base, BPW)],
                              wsems[e]).wait()
    pltpu.make_async_copy(eemb_v, eemb_hbm.at[pl.ds(base, BPW)],
                          sems[0]).wait()
    pltpu.make_async_copy(pemb_v, praw_hbm.at[pl.ds(base, BPW)],
                          sems[1]).wait()


def _sc_entry(nbrT_hbm, sidx_hbm, eidx_hbm, pidx_hbm, estart_hbm, eend_hbm,
              epath_hbm, agg_hbm, eemb_hbm, praw_hbm,
              idx_v, eidx_v, pidx_v,
              n0, n1, n2, n3,
              c0, c1, c2, c3,
              r0, r1, r2, r3,
              out_v, eemb_v, pemb_v,
              w0, w1, w2, w3,
              d0, d1, d2, d3, sem_m, sem_p):
    _sc_body(nbrT_hbm, sidx_hbm, eidx_hbm, pidx_hbm, estart_hbm, eend_hbm,
             epath_hbm, agg_hbm, eemb_hbm, praw_hbm,
             idx_v, eidx_v, pidx_v, (n0, n1, n2, n3), (c0, c1, c2, c3),
             (r0, r1, r2, r3),
             out_v, eemb_v, pemb_v, (w0, w1, w2, w3), (d0, d1, d2, d3),
             sem_m, sem_p)


_sc_gather = functools.partial(
    pl.kernel,
    out_type=(
        jax.ShapeDtypeStruct((NUM_ETYPES, BATCH, EMBED_DIM), jnp.float32),
        jax.ShapeDtypeStruct((BATCH, EMBED_DIM), jnp.float32),
        jax.ShapeDtypeStruct((BATCH, EMBED_DIM), jnp.float32),
    ),
    mesh=plsc.VectorSubcoreMesh(
        core_axis_name="c", subcore_axis_name="s", num_cores=NC,
        num_subcores=NS),
    compiler_params=pltpu.CompilerParams(needs_layout_passes=False),
    scratch_types=(
        [pltpu.VMEM((BPW,), jnp.int32)] * 3
        + [pltpu.VMEM((NUM_ETYPES, NEI, 128), jnp.int32)] * KB
        + [pltpu.VMEM((ROWS,), jnp.int32)] * KB
        + [pltpu.VMEM((ROWS, EMBED_DIM), jnp.float32)] * KB
        + [pltpu.VMEM((NUM_ETYPES, BPW, EMBED_DIM), jnp.float32)]
        + [pltpu.VMEM((BPW, EMBED_DIM), jnp.float32)] * 2
        + [pltpu.SemaphoreType.DMA] * (2 * KB + 2)
    ),
)(_sc_entry)


def _tc_body(agg_ref, eemb_ref, praw_ref, W1_ref, b1_ref, W2_ref, b2_ref,
             out_ref):
    f32 = jnp.float32
    hi = lax.Precision.HIGHEST
    W1 = W1_ref[...]
    b1 = b1_ref[...]
    acc = jnp.broadcast_to(b2_ref[...], (BATCH, EMBED_DIM))
    for e in range(NUM_ETYPES):
        h = lax.dot_general(agg_ref[e], W1, (((1,), (0,)), ((), ())),
                            precision=hi, preferred_element_type=f32) + b1
        W2e = W2_ref[pl.ds(e * EMBED_DIM, EMBED_DIM), :]
        acc = acc + lax.dot_general(h, W2e, (((1,), (0,)), ((), ())),
                                    precision=hi, preferred_element_type=f32)
    p = jax.nn.sigmoid(praw_ref[...])
    m = acc * eemb_ref[...] * p
    out_ref[...] = jax.nn.sigmoid(jnp.sum(m, axis=1, keepdims=True))


_tc_dense = pl.pallas_call(
    _tc_body,
    out_shape=jax.ShapeDtypeStruct((BATCH, 1), jnp.float32),
)


def kernel(neighbors, start_node, end_node, path, embeds_start, embeds_end,
           embeds_path, W1, b1, W2, b2):
    nbrT = jnp.transpose(neighbors, (1, 2, 0))
    agg, eemb, praw = _sc_gather(
        nbrT, start_node.astype(jnp.int32), end_node.astype(jnp.int32),
        path.astype(jnp.int32), embeds_start, embeds_end, embeds_path)
    out = _tc_dense(agg, eemb, praw, W1, b1.reshape(1, EMBED_DIM), W2,
                    b2.reshape(1, EMBED_DIM))
    return out.reshape(BATCH)
